# baseline (speedup 1.0000x reference)
import functools

import jax
import jax.numpy as jnp
from jax import lax
from jax.experimental import pallas as pl
from jax.experimental.pallas import tpu as pltpu
from jax.experimental.pallas import tpu_sc as plsc

_N = 1414
_SIZE = 64
_TWO_N_P1 = 2 * _N + 1
_G = 128


@functools.lru_cache(maxsize=None)
def _make_sc_gather(B: int):
    info = plsc.get_sparse_core_info()
    NC, NS, L = info.num_cores, info.num_subcores, info.num_lanes
    NW = NC * NS
    assert B % (NW * _G) == 0
    b_per_w = B // NW
    n_g = b_per_w // _G
    mesh = plsc.VectorSubcoreMesh(core_axis_name="c", subcore_axis_name="s")

    @functools.partial(
        pl.kernel,
        mesh=mesh,
        out_type=jax.ShapeDtypeStruct((B, _SIZE), jnp.float32),
        compiler_params=pltpu.CompilerParams(use_tc_tiling_on_sc=False),
        scratch_types=[
            pltpu.VMEM((b_per_w,), jnp.int32),
            pltpu.VMEM((b_per_w,), jnp.int32),
            pltpu.VMEM((b_per_w,), jnp.int32),
            pltpu.VMEM((_G, _SIZE), jnp.float32),
            pltpu.SemaphoreType.DMA,
        ],
    )
    def gather_kernel(i_hbm, j_hbm, w_hbm, out_hbm, iv, jv, kv, rows, gsem):
        wid = lax.axis_index("s") * NC + lax.axis_index("c")
        base = wid * b_per_w
        pltpu.sync_copy(i_hbm.at[pl.ds(base, b_per_w)], iv)
        pltpu.sync_copy(j_hbm.at[pl.ds(base, b_per_w)], jv)

        def compute(s, carry):
            ii = iv[pl.ds(s * L, L)]
            jj = jv[pl.ds(s * L, L)]
            lo = jnp.minimum(ii, jj)
            hi = jnp.maximum(ii, jj)
            prod = (_TWO_N_P1 - lo) * lo
            kv[pl.ds(s * L, L)] = (prod >> 1) + (hi - lo)
            return carry

        lax.fori_loop(0, b_per_w // L, compute, 0, unroll=4)

        def step(g, carry):
            pltpu.async_copy(
                w_hbm.at[kv.at[pl.ds(g * _G, _G)]], rows, gsem).wait()
            pltpu.sync_copy(rows, out_hbm.at[pl.ds(base + g * _G, _G)])
            return carry

        lax.fori_loop(0, n_g, step, 0)

    return gather_kernel


def kernel(i, j, w):
    shape = i.shape
    B = i.size
    i_flat = i.reshape(B).astype(jnp.int32)
    j_flat = j.reshape(B).astype(jnp.int32)
    out = _make_sc_gather(B)(i_flat, j_flat, w)
    return out.reshape(shape + (_SIZE,))


# speedup vs baseline: 1.0781x; 1.0781x over previous
import functools

import jax
import jax.numpy as jnp
from jax import lax
from jax.experimental import pallas as pl
from jax.experimental.pallas import tpu as pltpu
from jax.experimental.pallas import tpu_sc as plsc

_N = 1414
_SIZE = 64
_TWO_N_P1 = 2 * _N + 1
_G = 128
_S = 4


@functools.lru_cache(maxsize=None)
def _make_sc_gather(B: int):
    info = plsc.get_sparse_core_info()
    NC, NS, L = info.num_cores, info.num_subcores, info.num_lanes
    NW = NC * NS
    assert B % (NW * _G) == 0
    b_per_w = B // NW
    n_g = b_per_w // _G
    n_rounds = n_g // _S
    assert n_g % _S == 0 and n_rounds % 2 == 0 and n_rounds >= 4
    mesh = plsc.VectorSubcoreMesh(core_axis_name="c", subcore_axis_name="s")

    @functools.partial(
        pl.kernel,
        mesh=mesh,
        out_type=jax.ShapeDtypeStruct((B, _SIZE), jnp.float32),
        compiler_params=pltpu.CompilerParams(use_tc_tiling_on_sc=False),
        scratch_types=[
            pltpu.VMEM((b_per_w,), jnp.int32),
            pltpu.VMEM((b_per_w,), jnp.int32),
            pltpu.VMEM((2 * _S, _G, _SIZE), jnp.float32),
            [pltpu.SemaphoreType.DMA] * (2 * _S),
            [pltpu.SemaphoreType.DMA] * (2 * _S),
        ],
    )
    def gather_kernel(i_hbm, j_hbm, w_hbm, out_hbm, kv, jv, rows,
                      gsems, ssems):
        wid = lax.axis_index("s") * NC + lax.axis_index("c")
        base = wid * b_per_w
        pltpu.sync_copy(i_hbm.at[pl.ds(base, b_per_w)], kv)
        pltpu.sync_copy(j_hbm.at[pl.ds(base, b_per_w)], jv)

        def compute(s, carry):
            ii = kv[pl.ds(s * L, L)]
            jj = jv[pl.ds(s * L, L)]
            lo = jnp.minimum(ii, jj)
            hi = jnp.maximum(ii, jj)
            prod = (_TWO_N_P1 - lo) * lo
            kv[pl.ds(s * L, L)] = (prod >> 1) + (hi - lo)
            return carry

        lax.fori_loop(0, b_per_w // L, compute, 0, unroll=4)

        def gather_desc(g, slot):
            return pltpu.make_async_copy(
                w_hbm.at[kv.at[pl.ds(g * _G, _G)]], rows.at[slot],
                gsems[slot])

        def store_desc(g, slot):
            return pltpu.make_async_copy(
                rows.at[slot], out_hbm.at[pl.ds(base + g * _G, _G)],
                ssems[slot])

        def fire_round(g0, bank):
            for s in range(_S):
                gather_desc(g0 + s, bank * _S + s).start()

        def process_round(g0, bank):
            for s in range(_S):
                gather_desc(g0 + s, bank * _S + s).wait()
                store_desc(g0 + s, bank * _S + s).start()

        def drain_stores(g0, bank):
            for s in range(_S):
                store_desc(g0 + s, bank * _S + s).wait()

        fire_round(0, 0)
        fire_round(_S, 1)

        def body(tt, carry):
            g0 = 2 * tt * _S
            process_round(g0, 0)
            drain_stores(g0, 0)
            fire_round(g0 + 2 * _S, 0)
            process_round(g0 + _S, 1)
            drain_stores(g0 + _S, 1)
            fire_round(g0 + 3 * _S, 1)
            return carry

        lax.fori_loop(0, (n_rounds - 2) // 2, body, 0)

        g0 = (n_rounds - 2) * _S
        process_round(g0, 0)
        drain_stores(g0, 0)
        process_round(g0 + _S, 1)
        drain_stores(g0 + _S, 1)

    return gather_kernel


def kernel(i, j, w):
    shape = i.shape
    B = i.size
    i_flat = i.reshape(B).astype(jnp.int32)
    j_flat = j.reshape(B).astype(jnp.int32)
    out = _make_sc_gather(B)(i_flat, j_flat, w)
    return out.reshape(shape + (_SIZE,))
